# baseline (speedup 1.0000x reference)
import functools

import jax
import jax.numpy as jnp
from jax import lax
from jax.experimental import pallas as pl
from jax.experimental.pallas import tpu as pltpu
import jax.experimental.pallas.tpu_sc as plsc

NN = 10000
NNP = 10008
HH = 128
NLAYERS = 6
NCORES = 2
NSUB = 16
NTILES = NCORES * NSUB
EC = 64
CHUNKS = 160
PASSES = 4
CPP = CHUNKS // PASSES
NBUF = 4
EPAD = NTILES * CHUNKS * EC
DEGW = 16


def _stage_idx_row(idx2d, j, idx1):
    for k in range(EC // 16):
        idx1[pl.ds(k * 16, 16)] = idx2d[j, pl.ds(k * 16, 16)]


def _sc_deg_body(zeros_hbm, dst_hbm, out_hbm, dst_v, didx0, didx1, ones_v,
                 acc_sh, ssem0, ssem1):
    cid = lax.axis_index("c")
    sid = lax.axis_index("s")
    tid = cid * NSUB + sid

    one16 = jnp.ones((16,), jnp.float32)

    def fill_ones(r, carry):
        ones_v[r, :] = one16
        return carry

    lax.fori_loop(0, EC, fill_ones, 0)

    @pl.when(sid == 0)
    def _():
        pltpu.sync_copy(zeros_hbm, acc_sh)

    pltpu.sync_copy(dst_hbm.at[tid], dst_v)
    plsc.subcore_barrier()

    didx = (didx0, didx1)
    ssem = (ssem0, ssem1)

    def body(jj, carry):
        for b in range(2):
            j = jj * 2 + b

            @pl.when(j > 0)
            def _():
                pltpu.make_async_copy(
                    ones_v, acc_sh.at[didx[1 - b]], ssem[1 - b]).wait()

            _stage_idx_row(dst_v, j, didx[b])
            pltpu.async_copy(ones_v, acc_sh.at[didx[b]], ssem[b], add=True)
        return carry

    lax.fori_loop(0, CHUNKS // 2, body, 0)
    pltpu.make_async_copy(ones_v, acc_sh.at[didx1], ssem1).wait()
    plsc.subcore_barrier()

    @pl.when(sid == 0)
    def _():
        pltpu.sync_copy(acc_sh, out_hbm.at[cid])


def _sc_agg_body(zeros_hbm, g_hbm, src_hbm, dst_hbm, out_hbm,
                 src_v, dst_v,
                 sidx0, sidx1, sidx2, sidx3, didx0, didx1, didx2, didx3,
                 rows0, rows1, rows2, rows3,
                 acc_sh,
                 gsem0, gsem1, gsem2, gsem3, ssem0, ssem1, ssem2, ssem3):
    cid = lax.axis_index("c")
    sid = lax.axis_index("s")
    tid = cid * NSUB + sid

    @pl.when(sid == 0)
    def _():
        pltpu.sync_copy(zeros_hbm, acc_sh)

    plsc.subcore_barrier()

    sidx = (sidx0, sidx1, sidx2, sidx3)
    didx = (didx0, didx1, didx2, didx3)
    rows = (rows0, rows1, rows2, rows3)
    gsem = (gsem0, gsem1, gsem2, gsem3)
    ssem = (ssem0, ssem1, ssem2, ssem3)

    def gwait(s):
        pltpu.make_async_copy(g_hbm.at[sidx[s]], rows[s], gsem[s]).wait()

    def swait(s):
        pltpu.make_async_copy(rows[s], acc_sh.at[didx[s]], ssem[s]).wait()

    for p in range(PASSES):
        pltpu.sync_copy(src_hbm.at[tid, pl.ds(p * CPP, CPP)], src_v)
        pltpu.sync_copy(dst_hbm.at[tid, pl.ds(p * CPP, CPP)], dst_v)
        for s in range(2):
            _stage_idx_row(src_v, s, sidx[s])
            pltpu.async_copy(g_hbm.at[sidx[s]], rows[s], gsem[s])

        def body(jj, carry):
            for b in range(4):
                j = jj * 4 + b
                s = b
                s2 = (b + 2) % 4

                @pl.when(j >= 2)
                def _():
                    swait(s2)

                @pl.when(j + 2 < CPP)
                def _():
                    _stage_idx_row(src_v, j + 2, sidx[s2])
                    pltpu.async_copy(g_hbm.at[sidx[s2]], rows[s2], gsem[s2])

                gwait(s)
                _stage_idx_row(dst_v, j, didx[s])
                pltpu.async_copy(rows[s], acc_sh.at[didx[s]], ssem[s],
                                 add=True)
            return carry

        lax.fori_loop(0, CPP // 4, body, 0)
        swait((CPP - 2) % 4)
        swait((CPP - 1) % 4)

    plsc.subcore_barrier()

    @pl.when(sid == 0)
    def _():
        pltpu.sync_copy(acc_sh, out_hbm.at[cid])


_SC_MESH = plsc.VectorSubcoreMesh(core_axis_name="c", subcore_axis_name="s",
                                  num_cores=NCORES, num_subcores=NSUB)

_sc_deg = pl.kernel(
    _sc_deg_body,
    out_type=jax.ShapeDtypeStruct((NCORES, NNP, DEGW), jnp.float32),
    mesh=_SC_MESH,
    scratch_types=[
        pltpu.VMEM((CHUNKS, EC), jnp.int32),
        pltpu.VMEM((EC,), jnp.int32),
        pltpu.VMEM((EC,), jnp.int32),
        pltpu.VMEM((EC, DEGW), jnp.float32),
        pltpu.VMEM_SHARED((NNP, DEGW), jnp.float32),
        pltpu.SemaphoreType.DMA,
        pltpu.SemaphoreType.DMA,
    ],
)

_sc_agg = pl.kernel(
    _sc_agg_body,
    out_type=jax.ShapeDtypeStruct((NCORES, NNP, HH), jnp.float32),
    mesh=_SC_MESH,
    scratch_types=(
        [pltpu.VMEM((CPP, EC), jnp.int32)] * 2
        + [pltpu.VMEM((EC,), jnp.int32)] * (2 * NBUF)
        + [pltpu.VMEM((EC, HH), jnp.float32)] * NBUF
        + [pltpu.VMEM_SHARED((NNP, HH), jnp.float32)]
        + [pltpu.SemaphoreType.DMA] * (2 * NBUF)
    ),
)


def _tc_pre_body(x_ref, degp_ref, win_ref, bin_ref, w1_ref, b1_ref,
                 h0_ref, g1_ref, dinv_ref):
    deg = degp_ref[0, :NN, 0:1] + degp_ref[1, :NN, 0:1] + 1.0
    dinv = lax.rsqrt(deg)
    dinv_ref[...] = dinv
    h0 = jnp.dot(x_ref[...], win_ref[...],
                 preferred_element_type=jnp.float32) + bin_ref[...]
    h0_ref[...] = h0
    t1 = jnp.dot(h0, w1_ref[...],
                 preferred_element_type=jnp.float32) + b1_ref[...]
    g1_ref[:NN, :] = t1 * dinv


_tc_pre = pl.pallas_call(
    _tc_pre_body,
    out_shape=(
        jax.ShapeDtypeStruct((NN, HH), jnp.float32),
        jax.ShapeDtypeStruct((NNP, HH), jnp.float32),
        jax.ShapeDtypeStruct((NN, 1), jnp.float32),
    ),
)


def _tc_stats_body(acc_ref, g_ref, dinv_ref, agg_ref, mu_ref, rstd_ref):
    agg = (acc_ref[0, :NN, :] + acc_ref[1, :NN, :]
           + g_ref[:NN, :]) * dinv_ref[...]
    agg_ref[...] = agg
    mu = jnp.mean(agg, axis=0, keepdims=True)
    d = agg - mu
    var = jnp.mean(d * d, axis=0, keepdims=True)
    mu_ref[...] = mu
    rstd_ref[...] = lax.rsqrt(var + 1e-5)


_tc_stats = pl.pallas_call(
    _tc_stats_body,
    out_shape=(
        jax.ShapeDtypeStruct((NN, HH), jnp.float32),
        jax.ShapeDtypeStruct((1, HH), jnp.float32),
        jax.ShapeDtypeStruct((1, HH), jnp.float32),
    ),
)


def _softmax6(lw):
    e = jnp.exp(lw - jnp.max(lw))
    return e / jnp.sum(e)


def _bn_res(agg_ref, mu_ref, rstd_ref, gamma_ref, beta_ref,
            xin_ref, hprev_ref, layer):
    hb = ((agg_ref[...] - mu_ref[...]) * rstd_ref[...]
          * gamma_ref[...] + beta_ref[...])
    h = jnp.maximum(hb, 0.0) + 0.2 * xin_ref[...]
    if layer > 0:
        h = h + 0.5 * hprev_ref[...]
    return h


def _tc_mid_body(agg_ref, mu_ref, rstd_ref, gamma_ref, beta_ref,
                 xin_ref, hprev_ref, oacc_ref, lw_ref, dinv_ref,
                 wn_ref, bn_ref,
                 h_ref, gn_ref, onew_ref, *, layer):
    h = _bn_res(agg_ref, mu_ref, rstd_ref, gamma_ref, beta_ref,
                xin_ref, hprev_ref, layer)
    h_ref[...] = h
    sm = _softmax6(lw_ref[...])
    if layer > 0:
        onew_ref[...] = oacc_ref[...] + h * sm[layer:layer + 1]
    else:
        onew_ref[...] = h * sm[0:1]
    t = jnp.dot(h, wn_ref[...], preferred_element_type=jnp.float32) + bn_ref[...]
    gn_ref[:NN, :] = t * dinv_ref[...]


def _make_tc_mid(layer):
    return pl.pallas_call(
        functools.partial(_tc_mid_body, layer=layer),
        out_shape=(
            jax.ShapeDtypeStruct((NN, HH), jnp.float32),
            jax.ShapeDtypeStruct((NNP, HH), jnp.float32),
            jax.ShapeDtypeStruct((NN, HH), jnp.float32),
        ),
        input_output_aliases={7: 2} if layer > 0 else {},
    )


_tc_mids = [_make_tc_mid(i) for i in range(NLAYERS - 1)]


def _tc_final_body(agg_ref, mu_ref, rstd_ref, gamma_ref, beta_ref,
                   xin_ref, hprev_ref, oacc_ref, lw_ref,
                   wout_ref, bout_ref, out_ref):
    h6 = _bn_res(agg_ref, mu_ref, rstd_ref, gamma_ref, beta_ref,
                 xin_ref, hprev_ref, NLAYERS - 1)
    sm = _softmax6(lw_ref[...])
    outx = oacc_ref[...] + h6 * sm[NLAYERS - 1:NLAYERS]
    logits = jnp.dot(outx, wout_ref[...],
                     preferred_element_type=jnp.float32) + bout_ref[...]
    mx = jnp.max(logits, axis=1, keepdims=True)
    z = logits - mx
    lse = jnp.log(jnp.sum(jnp.exp(z), axis=1, keepdims=True))
    out_ref[...] = z - lse


def _make_tc_final(ncls):
    return pl.pallas_call(
        _tc_final_body,
        out_shape=jax.ShapeDtypeStruct((NN, ncls), jnp.float32),
    )


def kernel(x, edge_index, params):
    ncls = params['out_fc_w'].shape[1]
    src = edge_index[0]
    dst = edge_index[1]
    npad_e = EPAD - src.shape[0]
    pad_idx = jnp.full((npad_e,), NN, jnp.int32)
    src3 = jnp.concatenate([src, pad_idx]).reshape(NTILES, CHUNKS, EC)
    dst3 = jnp.concatenate([dst, pad_idx]).reshape(NTILES, CHUNKS, EC)

    zeros_deg = jnp.zeros((NNP, DEGW), jnp.float32)
    zeros_agg = jnp.zeros((NNP, HH), jnp.float32)
    degp = _sc_deg(zeros_deg, dst3)
    h0, g, dinv = _tc_pre(x, degp,
                          params['input_fc_w'], params['input_fc_b'],
                          params['conv_w'][0], params['conv_b'][0])

    hprev = h0
    oacc = h0
    lw = params['layer_weights']
    for i in range(NLAYERS - 1):
        acc = _sc_agg(zeros_agg, g, src3, dst3)
        agg, mu, rstd = _tc_stats(acc, g, dinv)
        hprev, g, oacc = _tc_mids[i](agg, mu, rstd,
                                     params['bn_gamma'][i],
                                     params['bn_beta'][i],
                                     h0, hprev, oacc, lw, dinv,
                                     params['conv_w'][i + 1],
                                     params['conv_b'][i + 1])

    acc = _sc_agg(zeros_agg, g, src3, dst3)
    agg, mu, rstd = _tc_stats(acc, g, dinv)
    out = _make_tc_final(ncls)(agg, mu, rstd,
                               params['bn_gamma'][NLAYERS - 1],
                               params['bn_beta'][NLAYERS - 1],
                               h0, hprev, oacc, lw,
                               params['out_fc_w'], params['out_fc_b'])
    return out


# speedup vs baseline: 1.1184x; 1.1184x over previous
import functools

import jax
import jax.numpy as jnp
from jax import lax
from jax.experimental import pallas as pl
from jax.experimental.pallas import tpu as pltpu
import jax.experimental.pallas.tpu_sc as plsc

NN = 10000
NNP = 10008
HH = 128
NLAYERS = 6
NCORES = 2
NSUB = 16
NTILES = NCORES * NSUB
EC = 128
CHUNKS = 80
PASSES = 2
CPP = CHUNKS // PASSES
NBUF = 2
EPAD = NTILES * CHUNKS * EC
DEGW = 16


def _stage_idx_row(idx2d, j, idx1):
    for k in range(EC // 16):
        idx1[pl.ds(k * 16, 16)] = idx2d[j, pl.ds(k * 16, 16)]


def _sc_deg_body(zeros_hbm, dst_hbm, out_hbm, dst_v, didx0, didx1, ones_v,
                 acc_sh, ssem0, ssem1):
    cid = lax.axis_index("c")
    sid = lax.axis_index("s")
    tid = cid * NSUB + sid

    one16 = jnp.ones((16,), jnp.float32)

    def fill_ones(r, carry):
        ones_v[r, :] = one16
        return carry

    lax.fori_loop(0, EC, fill_ones, 0)

    @pl.when(sid == 0)
    def _():
        pltpu.sync_copy(zeros_hbm, acc_sh)

    pltpu.sync_copy(dst_hbm.at[tid], dst_v)
    plsc.subcore_barrier()

    didx = (didx0, didx1)
    ssem = (ssem0, ssem1)

    def body(jj, carry):
        for b in range(2):
            j = jj * 2 + b

            @pl.when(j > 0)
            def _():
                pltpu.make_async_copy(
                    ones_v, acc_sh.at[didx[1 - b]], ssem[1 - b]).wait()

            _stage_idx_row(dst_v, j, didx[b])
            pltpu.async_copy(ones_v, acc_sh.at[didx[b]], ssem[b], add=True)
        return carry

    lax.fori_loop(0, CHUNKS // 2, body, 0)
    pltpu.make_async_copy(ones_v, acc_sh.at[didx1], ssem1).wait()
    plsc.subcore_barrier()

    @pl.when(sid == 0)
    def _():
        pltpu.sync_copy(acc_sh, out_hbm.at[cid])


def _sc_agg_body(zeros_hbm, g_hbm, src_hbm, dst_hbm, out_hbm,
                 src_v, dst_v, sidx0, sidx1, didx0, didx1, rows0, rows1,
                 acc_sh, gsem0, gsem1, ssem0, ssem1):
    cid = lax.axis_index("c")
    sid = lax.axis_index("s")
    tid = cid * NSUB + sid

    @pl.when(sid == 0)
    def _():
        pltpu.sync_copy(zeros_hbm, acc_sh)

    plsc.subcore_barrier()

    sidx = (sidx0, sidx1)
    didx = (didx0, didx1)
    rows = (rows0, rows1)
    gsem = (gsem0, gsem1)
    ssem = (ssem0, ssem1)

    for p in range(PASSES):
        pltpu.sync_copy(src_hbm.at[tid, pl.ds(p * CPP, CPP)], src_v)
        pltpu.sync_copy(dst_hbm.at[tid, pl.ds(p * CPP, CPP)], dst_v)
        _stage_idx_row(src_v, 0, sidx0)
        pltpu.async_copy(g_hbm.at[sidx0], rows0, gsem0)

        def body(jj, carry):
            for b in range(2):
                j = jj * 2 + b
                pltpu.make_async_copy(g_hbm.at[sidx[b]], rows[b],
                                      gsem[b]).wait()

                @pl.when(j > 0)
                def _():
                    pltpu.make_async_copy(
                        rows[1 - b], acc_sh.at[didx[1 - b]],
                        ssem[1 - b]).wait()

                @pl.when(j < CPP - 1)
                def _():
                    _stage_idx_row(src_v, j + 1, sidx[1 - b])
                    pltpu.async_copy(g_hbm.at[sidx[1 - b]], rows[1 - b],
                                     gsem[1 - b])

                _stage_idx_row(dst_v, j, didx[b])
                pltpu.async_copy(rows[b], acc_sh.at[didx[b]], ssem[b],
                                 add=True)
            return carry

        lax.fori_loop(0, CPP // 2, body, 0)
        pltpu.make_async_copy(rows1, acc_sh.at[didx1], ssem1).wait()

    plsc.subcore_barrier()

    @pl.when(sid == 0)
    def _():
        pltpu.sync_copy(acc_sh, out_hbm.at[cid])


_SC_MESH = plsc.VectorSubcoreMesh(core_axis_name="c", subcore_axis_name="s",
                                  num_cores=NCORES, num_subcores=NSUB)

_sc_deg = pl.kernel(
    _sc_deg_body,
    out_type=jax.ShapeDtypeStruct((NCORES, NNP, DEGW), jnp.float32),
    mesh=_SC_MESH,
    scratch_types=[
        pltpu.VMEM((CHUNKS, EC), jnp.int32),
        pltpu.VMEM((EC,), jnp.int32),
        pltpu.VMEM((EC,), jnp.int32),
        pltpu.VMEM((EC, DEGW), jnp.float32),
        pltpu.VMEM_SHARED((NNP, DEGW), jnp.float32),
        pltpu.SemaphoreType.DMA,
        pltpu.SemaphoreType.DMA,
    ],
)

_sc_agg = pl.kernel(
    _sc_agg_body,
    out_type=jax.ShapeDtypeStruct((NCORES, NNP, HH), jnp.float32),
    mesh=_SC_MESH,
    scratch_types=(
        [pltpu.VMEM((CPP, EC), jnp.int32)] * 2
        + [pltpu.VMEM((EC,), jnp.int32)] * (2 * NBUF)
        + [pltpu.VMEM((EC, HH), jnp.float32)] * NBUF
        + [pltpu.VMEM_SHARED((NNP, HH), jnp.float32)]
        + [pltpu.SemaphoreType.DMA] * (2 * NBUF)
    ),
)


def _tc_pre_body(x_ref, degp_ref, win_ref, bin_ref, w1_ref, b1_ref,
                 h0_ref, g1_ref, dinv_ref):
    deg = degp_ref[0, :NN, 0:1] + degp_ref[1, :NN, 0:1] + 1.0
    dinv = lax.rsqrt(deg)
    dinv_ref[...] = dinv
    h0 = jnp.dot(x_ref[...], win_ref[...],
                 preferred_element_type=jnp.float32) + bin_ref[...]
    h0_ref[...] = h0
    t1 = jnp.dot(h0, w1_ref[...],
                 preferred_element_type=jnp.float32) + b1_ref[...]
    g1_ref[:NN, :] = t1 * dinv


_tc_pre = pl.pallas_call(
    _tc_pre_body,
    out_shape=(
        jax.ShapeDtypeStruct((NN, HH), jnp.float32),
        jax.ShapeDtypeStruct((NNP, HH), jnp.float32),
        jax.ShapeDtypeStruct((NN, 1), jnp.float32),
    ),
)


def _tc_stats_body(acc_ref, g_ref, dinv_ref, agg_ref, mu_ref, rstd_ref):
    agg = (acc_ref[0, :NN, :] + acc_ref[1, :NN, :]
           + g_ref[:NN, :]) * dinv_ref[...]
    agg_ref[...] = agg
    mu = jnp.mean(agg, axis=0, keepdims=True)
    d = agg - mu
    var = jnp.mean(d * d, axis=0, keepdims=True)
    mu_ref[...] = mu
    rstd_ref[...] = lax.rsqrt(var + 1e-5)


_tc_stats = pl.pallas_call(
    _tc_stats_body,
    out_shape=(
        jax.ShapeDtypeStruct((NN, HH), jnp.float32),
        jax.ShapeDtypeStruct((1, HH), jnp.float32),
        jax.ShapeDtypeStruct((1, HH), jnp.float32),
    ),
)


def _softmax6(lw):
    e = jnp.exp(lw - jnp.max(lw))
    return e / jnp.sum(e)


def _bn_res(agg_ref, mu_ref, rstd_ref, gamma_ref, beta_ref,
            xin_ref, hprev_ref, layer):
    hb = ((agg_ref[...] - mu_ref[...]) * rstd_ref[...]
          * gamma_ref[...] + beta_ref[...])
    h = jnp.maximum(hb, 0.0) + 0.2 * xin_ref[...]
    if layer > 0:
        h = h + 0.5 * hprev_ref[...]
    return h


def _tc_mid_body(agg_ref, mu_ref, rstd_ref, gamma_ref, beta_ref,
                 xin_ref, hprev_ref, oacc_ref, lw_ref, dinv_ref,
                 wn_ref, bn_ref,
                 h_ref, gn_ref, onew_ref, *, layer):
    h = _bn_res(agg_ref, mu_ref, rstd_ref, gamma_ref, beta_ref,
                xin_ref, hprev_ref, layer)
    h_ref[...] = h
    sm = _softmax6(lw_ref[...])
    if layer > 0:
        onew_ref[...] = oacc_ref[...] + h * sm[layer:layer + 1]
    else:
        onew_ref[...] = h * sm[0:1]
    t = jnp.dot(h, wn_ref[...], preferred_element_type=jnp.float32) + bn_ref[...]
    gn_ref[:NN, :] = t * dinv_ref[...]


def _make_tc_mid(layer):
    return pl.pallas_call(
        functools.partial(_tc_mid_body, layer=layer),
        out_shape=(
            jax.ShapeDtypeStruct((NN, HH), jnp.float32),
            jax.ShapeDtypeStruct((NNP, HH), jnp.float32),
            jax.ShapeDtypeStruct((NN, HH), jnp.float32),
        ),
        input_output_aliases={7: 2} if layer > 0 else {},
    )


_tc_mids = [_make_tc_mid(i) for i in range(NLAYERS - 1)]


def _tc_final_body(agg_ref, mu_ref, rstd_ref, gamma_ref, beta_ref,
                   xin_ref, hprev_ref, oacc_ref, lw_ref,
                   wout_ref, bout_ref, out_ref):
    h6 = _bn_res(agg_ref, mu_ref, rstd_ref, gamma_ref, beta_ref,
                 xin_ref, hprev_ref, NLAYERS - 1)
    sm = _softmax6(lw_ref[...])
    outx = oacc_ref[...] + h6 * sm[NLAYERS - 1:NLAYERS]
    logits = jnp.dot(outx, wout_ref[...],
                     preferred_element_type=jnp.float32) + bout_ref[...]
    mx = jnp.max(logits, axis=1, keepdims=True)
    z = logits - mx
    lse = jnp.log(jnp.sum(jnp.exp(z), axis=1, keepdims=True))
    out_ref[...] = z - lse


def _make_tc_final(ncls):
    return pl.pallas_call(
        _tc_final_body,
        out_shape=jax.ShapeDtypeStruct((NN, ncls), jnp.float32),
    )


def kernel(x, edge_index, params):
    ncls = params['out_fc_w'].shape[1]
    src = edge_index[0]
    dst = edge_index[1]
    npad_e = EPAD - src.shape[0]
    pad_idx = jnp.full((npad_e,), NN, jnp.int32)
    src3 = jnp.concatenate([src, pad_idx]).reshape(NTILES, CHUNKS, EC)
    dst3 = jnp.concatenate([dst, pad_idx]).reshape(NTILES, CHUNKS, EC)

    zeros_deg = jnp.zeros((NNP, DEGW), jnp.float32)
    zeros_agg = jnp.zeros((NNP, HH), jnp.float32)
    degp = _sc_deg(zeros_deg, dst3)
    h0, g, dinv = _tc_pre(x, degp,
                          params['input_fc_w'], params['input_fc_b'],
                          params['conv_w'][0], params['conv_b'][0])

    hprev = h0
    oacc = h0
    lw = params['layer_weights']
    for i in range(NLAYERS - 1):
        acc = _sc_agg(zeros_agg, g, src3, dst3)
        agg, mu, rstd = _tc_stats(acc, g, dinv)
        hprev, g, oacc = _tc_mids[i](agg, mu, rstd,
                                     params['bn_gamma'][i],
                                     params['bn_beta'][i],
                                     h0, hprev, oacc, lw, dinv,
                                     params['conv_w'][i + 1],
                                     params['conv_b'][i + 1])

    acc = _sc_agg(zeros_agg, g, src3, dst3)
    agg, mu, rstd = _tc_stats(acc, g, dinv)
    out = _make_tc_final(ncls)(agg, mu, rstd,
                               params['bn_gamma'][NLAYERS - 1],
                               params['bn_beta'][NLAYERS - 1],
                               h0, hprev, oacc, lw,
                               params['out_fc_w'], params['out_fc_b'])
    return out
